# baseline (speedup 1.0000x reference)
import jax
import jax.numpy as jnp
from jax.experimental import pallas as pl
from jax.experimental.pallas import tpu as pltpu

_HIDDEN = 1024
_E = 16
_FFN = 4096
_NTOK = 256
_FC = 1024
_NF = _FFN // _FC
_STEPS = _E * _NF
_NBUF = 4


def _issue(w1_hbm, w2_hbm, w1_buf, w2_buf, sem1, sem2, t):
    e = t // _NF
    f = t % _NF
    slot = t % _NBUF
    pltpu.make_async_copy(
        w1_hbm.at[e, pl.ds(f * _FC, _FC), :], w1_buf.at[slot], sem1.at[slot]
    ).start()
    pltpu.make_async_copy(
        w2_hbm.at[e, :, pl.ds(f * _FC, _FC)], w2_buf.at[slot], sem2.at[slot]
    ).start()


def _moe_body(x_ref, wr_ref, b1_ref, b2_ref, w1_hbm, w2_hbm, out_ref,
              wts_ref, w1_buf, w2_buf, sem1, sem2):
    s = pl.program_id(0)
    e = s // _NF
    f = s % _NF
    slot = s % _NBUF

    lane = jax.lax.broadcasted_iota(jnp.int32, (_NTOK, _E), 1)

    @pl.when(s == 0)
    def _prologue():
        for t in range(_NBUF - 1):
            _issue(w1_hbm, w2_hbm, w1_buf, w2_buf, sem1, sem2, t)

        logits = jax.lax.dot_general(
            x_ref[...], wr_ref[...], (((1,), (1,)), ((), ())),
            preferred_element_type=jnp.float32,
        )
        m1 = jnp.max(logits, axis=1, keepdims=True)
        i1 = jnp.min(jnp.where(logits == m1, lane, _E), axis=1, keepdims=True)
        masked = jnp.where(lane == i1, -jnp.inf, logits)
        m2 = jnp.max(masked, axis=1, keepdims=True)
        i2 = jnp.min(jnp.where(masked == m2, lane, _E), axis=1, keepdims=True)
        t = jnp.exp(m2 - m1)
        p1 = 1.0 / (1.0 + t)
        p2 = t / (1.0 + t)
        wts_ref[...] = jnp.where(lane == i1, p1, 0.0) + jnp.where(lane == i2, p2, 0.0)

    @pl.when(s + _NBUF - 1 < _STEPS)
    def _prefetch():
        _issue(w1_hbm, w2_hbm, w1_buf, w2_buf, sem1, sem2, s + _NBUF - 1)

    pltpu.make_async_copy(
        w1_hbm.at[0, pl.ds(0, _FC), :], w1_buf.at[slot], sem1.at[slot]
    ).wait()
    pltpu.make_async_copy(
        w2_hbm.at[0, :, pl.ds(0, _FC)], w2_buf.at[slot], sem2.at[slot]
    ).wait()

    xb = x_ref[...].astype(jnp.bfloat16)
    h = jax.lax.dot_general(
        xb, w1_buf[slot].astype(jnp.bfloat16), (((1,), (1,)), ((), ())),
        preferred_element_type=jnp.float32,
    )
    h = h + b1_ref[0]
    a = 0.5 * h * (1.0 + jax.lax.erf(h * 0.7071067811865476))
    o = jax.lax.dot_general(
        a.astype(jnp.bfloat16), w2_buf[slot].astype(jnp.bfloat16),
        (((1,), (1,)), ((), ())),
        preferred_element_type=jnp.float32,
    )
    o = jnp.where(f == 0, o + b2_ref[0], o)
    wcol = jnp.sum(wts_ref[...] * (lane == e).astype(jnp.float32),
                   axis=1, keepdims=True)
    contrib = wcol * o

    @pl.when(s == 0)
    def _init():
        out_ref[...] = contrib

    @pl.when(s > 0)
    def _acc():
        out_ref[...] += contrib


def kernel(x, Wr, W1, b1, W2, b2):
    B, S, D = x.shape
    xf = x.reshape(B * S, D)
    b1r = b1.reshape(_STEPS, 1, _FC)
    b2r = b2.reshape(_E, 1, _HIDDEN)
    out = pl.pallas_call(
        _moe_body,
        grid=(_STEPS,),
        in_specs=[
            pl.BlockSpec((_NTOK, _HIDDEN), lambda s: (0, 0)),
            pl.BlockSpec((_E, _HIDDEN), lambda s: (0, 0)),
            pl.BlockSpec((1, 1, _FC), lambda s: (s, 0, 0)),
            pl.BlockSpec((1, 1, _HIDDEN), lambda s: (s // _NF, 0, 0)),
            pl.BlockSpec(memory_space=pltpu.HBM),
            pl.BlockSpec(memory_space=pltpu.HBM),
        ],
        out_specs=pl.BlockSpec((_NTOK, _HIDDEN), lambda s: (0, 0)),
        out_shape=jax.ShapeDtypeStruct((_NTOK, _HIDDEN), jnp.float32),
        scratch_shapes=[
            pltpu.VMEM((_NTOK, _E), jnp.float32),
            pltpu.VMEM((_NBUF, _FC, _HIDDEN), jnp.float32),
            pltpu.VMEM((_NBUF, _HIDDEN, _FC), jnp.float32),
            pltpu.SemaphoreType.DMA((_NBUF,)),
            pltpu.SemaphoreType.DMA((_NBUF,)),
        ],
        compiler_params=pltpu.CompilerParams(
            dimension_semantics=("arbitrary",),
        ),
    )(xf, Wr, b1r, b2r, W1, W2)
    return out.reshape(B, S, D)
